# Initial kernel scaffold; baseline (speedup 1.0000x reference)
import jax
import jax.numpy as jnp
from jax.experimental import pallas as pl


def kernel(pos, batch, pos_non_manifold, pos_non_manifold_batch, latents, W_in, b_in, W1, b1, W2, b2, W_out, b_out):
    raise NotImplementedError("write your pallas kernel here")


# speedup vs baseline: 4.7320x; 4.7320x over previous
import functools

import jax
import jax.numpy as jnp
from jax import lax
from jax.experimental import pallas as pl
from jax.experimental.pallas import tpu as pltpu
from jax.experimental.pallas import tpu_sc as plsc

N_S, N_T, LAT, KNN = 4096, 16384, 128, 16
NG = N_S // 16
TT = 128
NW = 32
N_E = N_T * KNN

_DEFAULT = lax.Precision.DEFAULT


def _a_body(lat_ref, pos_ref, wl_ref, wp_ref, bin_ref, a_ref):
    A = lax.dot_general(lat_ref[...], wl_ref[...], (((1,), (0,)), ((), ())),
                        precision=_DEFAULT)
    p = pos_ref[...]
    wp = wp_ref[...]
    P = (p[:, 0:1] * wp[0:1, :] + p[:, 1:2] * wp[1:2, :]
         + p[:, 2:3] * wp[2:3, :])
    a_ref[...] = A - P + bin_ref[...]


def _compute_a(latents, pos, w_lat, w_pos, b_in):
    return pl.pallas_call(
        _a_body,
        out_shape=jax.ShapeDtypeStruct((N_S, LAT), jnp.float32),
    )(latents, pos, w_lat, w_pos, b_in.reshape(1, LAT))


def _knn_body(t_ref, s_ref, st_ref, d2_ref, fidx_ref):
    i = pl.program_id(0)
    t = t_ref[...]
    s = s_ref[...]
    st = st_ref[...]
    M = lax.dot_general(t, s, (((1,), (1,)), ((), ())), precision=_DEFAULT)
    t0, t1, t2 = t[:, 0:1], t[:, 1:2], t[:, 2:3]
    tt = (t0 * t0 + t2 * t2) + t1 * t1
    s0, s1, s2 = st[0:1, :], st[1:2, :], st[2:3, :]
    ss = (s0 * s0 + s2 * s2) + s1 * s1
    d2 = (tt - 2.0 * M) + ss
    d2_ref[...] = d2
    G = jnp.min(d2.reshape(TT, NG, 16), axis=2)
    giota = lax.broadcasted_iota(jnp.int32, (TT, NG), 1)
    gids = []
    for _ in range(KNN):
        v = jnp.min(G, axis=1, keepdims=True)
        eq = G == v
        gid = jnp.min(jnp.where(eq, giota, NG), axis=1, keepdims=True)
        G = jnp.where(giota == gid, jnp.inf, G)
        gids.append(gid)
    gid16 = jnp.concatenate(gids, axis=1)
    rows = i * TT + lax.broadcasted_iota(jnp.int32, (TT, 1), 0)
    fidx_ref[...] = rows * NG + gid16


def _knn(pnm, pos, pos_t):
    return pl.pallas_call(
        _knn_body,
        grid=(N_T // TT,),
        in_specs=[pl.BlockSpec((TT, 3), lambda i: (i, 0)),
                  pl.BlockSpec((N_S, 3), lambda i: (0, 0)),
                  pl.BlockSpec((3, N_S), lambda i: (0, 0))],
        out_specs=[pl.BlockSpec((TT, N_S), lambda i: (i, 0)),
                   pl.BlockSpec((TT, KNN), lambda i: (i, 0))],
        out_shape=[jax.ShapeDtypeStruct((N_T, N_S), jnp.float32),
                   jax.ShapeDtypeStruct((N_T, KNN), jnp.int32)],
    )(pnm, pos, pos_t)


def _make_sc_gather(n_rows_table, d, n_idx, chunk_ir):
    ir_per_w = n_idx // 128 // NW

    def body(table, idx2d, out, idxv, rowsv, sem):
        wid = lax.axis_index("s") * 2 + lax.axis_index("c")
        irbase = wid * ir_per_w
        pltpu.sync_copy(idx2d.at[pl.ds(irbase, ir_per_w)], idxv)

        def chunk(ci, carry):
            handles = []
            for j in range(chunk_ir):
                handles.append(pltpu.async_copy(
                    table.at[idxv.at[ci * chunk_ir + j]],
                    rowsv.at[pl.ds(j * 128, 128)], sem))
            for h in handles:
                h.wait()
            rowstart = (irbase + ci * chunk_ir) * 128
            pltpu.sync_copy(rowsv, out.at[pl.ds(rowstart, chunk_ir * 128)])
            return carry

        lax.fori_loop(0, ir_per_w // chunk_ir, chunk, 0)

    return functools.partial(
        pl.kernel,
        out_type=jax.ShapeDtypeStruct((n_idx, d), jnp.float32),
        mesh=plsc.VectorSubcoreMesh(core_axis_name="c", subcore_axis_name="s"),
        compiler_params=pltpu.CompilerParams(use_tc_tiling_on_sc=False),
        scratch_types=[pltpu.VMEM((ir_per_w, 128), jnp.int32),
                       pltpu.VMEM((chunk_ir * 128, d), jnp.float32),
                       pltpu.SemaphoreType.DMA],
    )(body)


def _sel_body(c_ref, fidx_ref, cols_ref):
    C = c_ref[...]
    gid16 = fidx_ref[...] & (NG - 1)
    pieces = [jnp.broadcast_to(gid16[:, k:k + 1], (TT, 16))
              for k in range(KNN)]
    gexp = jnp.concatenate(pieces, axis=1)
    jiota = lax.broadcasted_iota(jnp.int32, (TT, NG), 1) & 15
    ccols = gexp * 16 + jiota
    outs = []
    for _ in range(KNN):
        v = jnp.min(C, axis=1, keepdims=True)
        eq = C == v
        col = jnp.min(jnp.where(eq, ccols, N_S), axis=1, keepdims=True)
        C = jnp.where(eq & (ccols == col), jnp.inf, C)
        outs.append(col)
    cols_ref[...] = jnp.concatenate(outs, axis=1)


def _select(c16, fidx):
    return pl.pallas_call(
        _sel_body,
        grid=(N_T // TT,),
        in_specs=[pl.BlockSpec((TT, NG), lambda i: (i, 0)),
                  pl.BlockSpec((TT, KNN), lambda i: (i, 0))],
        out_specs=pl.BlockSpec((TT, KNN), lambda i: (i, 0)),
        out_shape=jax.ShapeDtypeStruct((N_T, KNN), jnp.int32),
    )(c16, fidx)


def _mlp_body(g_ref, t_ref, wp_ref, w1_ref, b1_ref, w2_ref, b2_ref, wo_ref,
              bo_ref, out_ref):
    t = t_ref[...]
    wp = wp_ref[...]
    q = (t[:, 0:1] * wp[0:1, :] + t[:, 1:2] * wp[1:2, :]
         + t[:, 2:3] * wp[2:3, :])
    h = g_ref[...].reshape(TT, KNN, LAT) + q[:, None, :]
    x = jnp.maximum(h, 0.0).reshape(TT * KNN, LAT)
    x = lax.dot_general(x, w1_ref[...], (((1,), (0,)), ((), ())),
                        precision=_DEFAULT) + b1_ref[...]
    x = jnp.maximum(x, 0.0)
    x = lax.dot_general(x, w2_ref[...], (((1,), (0,)), ((), ())),
                        precision=_DEFAULT) + b2_ref[...]
    y = lax.dot_general(x, wo_ref[...], (((1,), (0,)), ((), ())),
                        precision=_DEFAULT) + bo_ref[...]
    out_ref[...] = y[:, 0]


def _mlp(g, pnm, w_pos, w1, b1, w2, b2, w_out, b_out):
    return pl.pallas_call(
        _mlp_body,
        grid=(N_T // TT,),
        in_specs=[pl.BlockSpec((TT * KNN, LAT), lambda i: (i, 0)),
                  pl.BlockSpec((TT, 3), lambda i: (i, 0)),
                  pl.BlockSpec((3, LAT), lambda i: (0, 0)),
                  pl.BlockSpec((LAT, LAT), lambda i: (0, 0)),
                  pl.BlockSpec((1, LAT), lambda i: (0, 0)),
                  pl.BlockSpec((LAT, LAT), lambda i: (0, 0)),
                  pl.BlockSpec((1, LAT), lambda i: (0, 0)),
                  pl.BlockSpec((LAT, 1), lambda i: (0, 0)),
                  pl.BlockSpec((1, 1), lambda i: (0, 0))],
        out_specs=pl.BlockSpec((TT * KNN,), lambda i: (i,)),
        out_shape=jax.ShapeDtypeStruct((N_E,), jnp.float32),
    )(g, pnm, w_pos, w1, b1.reshape(1, LAT), w2, b2.reshape(1, LAT),
      w_out, b_out.reshape(1, 1))


def kernel(pos, batch, pos_non_manifold, pos_non_manifold_batch, latents,
           W_in, b_in, W1, b1, W2, b2, W_out, b_out):
    w_lat = W_in[:LAT]
    w_pos = W_in[LAT:]
    a = _compute_a(latents, pos, w_lat, w_pos, b_in)
    d2, fidx = _knn(pos_non_manifold, pos, pos.T)
    c = _make_sc_gather(N_T * NG, 16, N_E, 8)(
        d2.reshape(N_T * NG, 16), fidx.reshape(N_E // 128, 128))
    cols = _select(c.reshape(N_T, NG), fidx)
    g = _make_sc_gather(N_S, LAT, N_E, 2)(
        a, cols.reshape(N_E // 128, 128))
    return _mlp(g, pos_non_manifold, w_pos, W1, b1, W2, b2, W_out, b_out)


# speedup vs baseline: 14.6612x; 3.0983x over previous
import functools

import jax
import jax.numpy as jnp
from jax import lax
from jax.experimental import pallas as pl
from jax.experimental.pallas import tpu as pltpu
from jax.experimental.pallas import tpu_sc as plsc

N_S, N_T, LAT, KNN = 4096, 16384, 128, 16
NG = N_S // 16
TT = 256
TSEL = 1024
TMLP = 256
NW = 32
N_E = N_T * KNN

_DEFAULT = lax.Precision.DEFAULT


def _a_body(lat_ref, pos_ref, wl_ref, wp_ref, bin_ref, a_ref):
    A = lax.dot_general(lat_ref[...], wl_ref[...], (((1,), (0,)), ((), ())),
                        precision=_DEFAULT)
    p = pos_ref[...]
    wp = wp_ref[...]
    P = (p[:, 0:1] * wp[0:1, :] + p[:, 1:2] * wp[1:2, :]
         + p[:, 2:3] * wp[2:3, :])
    a_ref[...] = A - P + bin_ref[...]


def _compute_a(latents, pos, w_lat, w_pos, b_in):
    return pl.pallas_call(
        _a_body,
        out_shape=jax.ShapeDtypeStruct((N_S, LAT), jnp.float32),
    )(latents, pos, w_lat, w_pos, b_in.reshape(1, LAT))


def _knn_body(t_ref, s_ref, st_ref, sp_ref, spt_ref, d2_ref, fidx_ref):
    i = pl.program_id(0)
    t = t_ref[...]
    s = s_ref[...]
    st = st_ref[...]
    M = lax.dot_general(t, s, (((1,), (1,)), ((), ())), precision=_DEFAULT)
    t0, t1, t2 = t[:, 0:1], t[:, 1:2], t[:, 2:3]
    tt = (t0 * t0 + t2 * t2) + t1 * t1
    s0, s1, s2 = st[0:1, :], st[1:2, :], st[2:3, :]
    ss = (s0 * s0 + s2 * s2) + s1 * s1
    d2 = (tt - 2.0 * M) + ss
    d2_ref[...] = d2
    Mp = lax.dot_general(t, sp_ref[...], (((1,), (1,)), ((), ())),
                         precision=_DEFAULT)
    spt = spt_ref[...]
    p0, p1, p2 = spt[0:1, :], spt[1:2, :], spt[2:3, :]
    ssp = (p0 * p0 + p2 * p2) + p1 * p1
    d2p = (tt - 2.0 * Mp) + ssp
    G = jnp.min(d2p.reshape(TT, 16, NG), axis=1)
    giota = lax.broadcasted_iota(jnp.int32, (TT, NG), 1)
    gids = []
    for _ in range(KNN):
        v = jnp.min(G, axis=1, keepdims=True)
        eq = G == v
        gid = jnp.min(jnp.where(eq, giota, NG), axis=1, keepdims=True)
        G = jnp.where(giota == gid, jnp.inf, G)
        gids.append(gid)
    gid16 = jnp.concatenate(gids, axis=1)
    rows = i * TT + lax.broadcasted_iota(jnp.int32, (TT, 1), 0)
    fidx_ref[...] = rows * NG + gid16


def _knn(pnm, pos, pos_t, pos_p, pos_pt):
    return pl.pallas_call(
        _knn_body,
        grid=(N_T // TT,),
        in_specs=[pl.BlockSpec((TT, 3), lambda i: (i, 0)),
                  pl.BlockSpec((N_S, 3), lambda i: (0, 0)),
                  pl.BlockSpec((3, N_S), lambda i: (0, 0)),
                  pl.BlockSpec((N_S, 3), lambda i: (0, 0)),
                  pl.BlockSpec((3, N_S), lambda i: (0, 0))],
        out_specs=[pl.BlockSpec((TT, N_S), lambda i: (i, 0)),
                   pl.BlockSpec((TT, KNN), lambda i: (i, 0))],
        out_shape=[jax.ShapeDtypeStruct((N_T, N_S), jnp.float32),
                   jax.ShapeDtypeStruct((N_T, KNN), jnp.int32)],
    )(pnm, pos, pos_t, pos_p, pos_pt)


def _make_sc_gather(n_rows_table, d, n_idx, chunk_ir):
    ir_per_w = n_idx // 128 // NW

    def body(table, idx2d, out, idxv, rowsv, sem):
        wid = lax.axis_index("s") * 2 + lax.axis_index("c")
        irbase = wid * ir_per_w
        pltpu.sync_copy(idx2d.at[pl.ds(irbase, ir_per_w)], idxv)

        def chunk(ci, carry):
            handles = []
            for j in range(chunk_ir):
                handles.append(pltpu.async_copy(
                    table.at[idxv.at[ci * chunk_ir + j]],
                    rowsv.at[pl.ds(j * 128, 128)], sem))
            for h in handles:
                h.wait()
            rowstart = (irbase + ci * chunk_ir) * 128
            pltpu.sync_copy(rowsv, out.at[pl.ds(rowstart, chunk_ir * 128)])
            return carry

        lax.fori_loop(0, ir_per_w // chunk_ir, chunk, 0)

    return functools.partial(
        pl.kernel,
        out_type=jax.ShapeDtypeStruct((n_idx, d), jnp.float32),
        mesh=plsc.VectorSubcoreMesh(core_axis_name="c", subcore_axis_name="s"),
        compiler_params=pltpu.CompilerParams(use_tc_tiling_on_sc=False),
        scratch_types=[pltpu.VMEM((ir_per_w, 128), jnp.int32),
                       pltpu.VMEM((chunk_ir * 128, d), jnp.float32),
                       pltpu.SemaphoreType.DMA],
    )(body)


def _sel_body(c_ref, fidx_ref, cols_ref):
    C = c_ref[...]
    gid16 = fidx_ref[...] & (NG - 1)
    siota = lax.broadcasted_iota(jnp.int32, (KNN, NG), 0)
    liota = lax.broadcasted_iota(jnp.int32, (KNN, NG), 1)
    expand = (siota == (liota >> 4)).astype(jnp.float32)
    gexp = lax.dot_general(gid16.astype(jnp.float32), expand,
                           (((1,), (0,)), ((), ())), precision=_DEFAULT)
    jiota = lax.broadcasted_iota(jnp.int32, (TSEL, NG), 1) & 15
    ccols = gexp * 16.0 + jiota.astype(jnp.float32)
    outs = []
    for _ in range(KNN):
        v = jnp.min(C, axis=1, keepdims=True)
        eq = C == v
        col = jnp.min(jnp.where(eq, ccols, float(N_S)), axis=1, keepdims=True)
        C = jnp.where(eq & (ccols == col), jnp.inf, C)
        outs.append(col)
    cols_ref[...] = jnp.concatenate(outs, axis=1).astype(jnp.int32)


def _select(c16, fidx):
    return pl.pallas_call(
        _sel_body,
        grid=(N_T // TSEL,),
        in_specs=[pl.BlockSpec((TSEL, NG), lambda i: (i, 0)),
                  pl.BlockSpec((TSEL, KNN), lambda i: (i, 0))],
        out_specs=pl.BlockSpec((TSEL, KNN), lambda i: (i, 0)),
        out_shape=jax.ShapeDtypeStruct((N_T, KNN), jnp.int32),
    )(c16, fidx)


def _mlp_body(g_ref, t_ref, wp_ref, w1_ref, b1_ref, w2_ref, b2_ref, wo_ref,
              bo_ref, out_ref):
    t = t_ref[...]
    wp = wp_ref[...]
    q = (t[:, 0:1] * wp[0:1, :] + t[:, 1:2] * wp[1:2, :]
         + t[:, 2:3] * wp[2:3, :])
    h = g_ref[...].reshape(TMLP, KNN, LAT) + q[:, None, :]
    x = jnp.maximum(h, 0.0).reshape(TMLP * KNN, LAT)
    x = lax.dot_general(x, w1_ref[...], (((1,), (0,)), ((), ())),
                        precision=_DEFAULT) + b1_ref[...]
    x = jnp.maximum(x, 0.0)
    x = lax.dot_general(x, w2_ref[...], (((1,), (0,)), ((), ())),
                        precision=_DEFAULT) + b2_ref[...]
    y = lax.dot_general(x, wo_ref[...], (((1,), (0,)), ((), ())),
                        precision=_DEFAULT) + bo_ref[...]
    out_ref[...] = y[:, 0]


def _mlp(g, pnm, w_pos, w1, b1, w2, b2, w_out, b_out):
    return pl.pallas_call(
        _mlp_body,
        grid=(N_T // TMLP,),
        in_specs=[pl.BlockSpec((TMLP * KNN, LAT), lambda i: (i, 0)),
                  pl.BlockSpec((TMLP, 3), lambda i: (i, 0)),
                  pl.BlockSpec((3, LAT), lambda i: (0, 0)),
                  pl.BlockSpec((LAT, LAT), lambda i: (0, 0)),
                  pl.BlockSpec((1, LAT), lambda i: (0, 0)),
                  pl.BlockSpec((LAT, LAT), lambda i: (0, 0)),
                  pl.BlockSpec((1, LAT), lambda i: (0, 0)),
                  pl.BlockSpec((LAT, 1), lambda i: (0, 0)),
                  pl.BlockSpec((1, 1), lambda i: (0, 0))],
        out_specs=pl.BlockSpec((TMLP * KNN,), lambda i: (i,)),
        out_shape=jax.ShapeDtypeStruct((N_E,), jnp.float32),
    )(g, pnm, w_pos, w1, b1.reshape(1, LAT), w2, b2.reshape(1, LAT),
      w_out, b_out.reshape(1, 1))


def kernel(pos, batch, pos_non_manifold, pos_non_manifold_batch, latents,
           W_in, b_in, W1, b1, W2, b2, W_out, b_out):
    w_lat = W_in[:LAT]
    w_pos = W_in[LAT:]
    a = _compute_a(latents, pos, w_lat, w_pos, b_in)
    cp = (jnp.arange(N_S, dtype=jnp.int32) % NG) * 16 + (
        jnp.arange(N_S, dtype=jnp.int32) // NG)
    pos_p = pos[cp]
    d2, fidx = _knn(pos_non_manifold, pos, pos.T, pos_p, pos_p.T)
    c = _make_sc_gather(N_T * NG, 16, N_E, 8)(
        d2.reshape(N_T * NG, 16), fidx.reshape(N_E // 128, 128))
    cols = _select(c.reshape(N_T, NG), fidx)
    g = _make_sc_gather(N_S, LAT, N_E, 2)(
        a, cols.reshape(N_E // 128, 128))
    return _mlp(g, pos_non_manifold, w_pos, W1, b1, W2, b2, W_out, b_out)
